# Initial kernel scaffold; baseline (speedup 1.0000x reference)
import jax
import jax.numpy as jnp
from jax.experimental import pallas as pl


def kernel(x, router_w, expert_w1, expert_w2):
    raise NotImplementedError("write your pallas kernel here")


# speedup vs baseline: 13.3921x; 13.3921x over previous
import functools

import jax
import jax.numpy as jnp
from jax import lax
from jax.experimental import pallas as pl
from jax.experimental.pallas import tpu as pltpu
from jax.experimental.pallas import tpu_sc as plsc

T = 2048
H = 768
E = 64
K = 2
F = 2048
BLK = 128
NPAIR = T * K
NBLK = 96
NPAD = NBLK * BLK
H2 = H // 2
NW = 32
PAIRS_W = NPAIR // NW
TOK_W = T // NW


def _router_body(x_ref, rw_ref, pos_ref, wp_ref, meta_ref):
    x = x_ref[...]
    rw = rw_ref[...]
    scores = lax.dot_general(
        x, rw, (((1,), (1,)), ((), ())),
        preferred_element_type=jnp.float32,
        precision=lax.Precision.DEFAULT)

    eio = lax.broadcasted_iota(jnp.int32, (T, E), 1)
    s1 = jnp.max(scores, axis=1, keepdims=True)
    a1 = jnp.min(jnp.where(scores == s1, eio, E), axis=1, keepdims=True)
    masked = jnp.where(eio == a1, jnp.float32(-jnp.inf), scores)
    s2 = jnp.max(masked, axis=1, keepdims=True)
    a2 = jnp.min(jnp.where(masked == s2, eio, E), axis=1, keepdims=True)
    ee = jnp.exp(s2 - s1)
    wa = 1.0 / (1.0 + ee)
    wb = ee / (1.0 + ee)

    e_pair = jnp.concatenate([a1, a2], axis=0)
    w_pair = jnp.concatenate([wa, wb], axis=0)

    pio = lax.broadcasted_iota(jnp.int32, (NPAIR, E), 1)
    onehot = (e_pair == pio).astype(jnp.float32)
    pref = onehot
    s = 1
    while s < NPAIR:
        pref = pref + jnp.concatenate(
            [jnp.zeros((s, E), jnp.float32), pref[:-s, :]], axis=0)
        s *= 2
    counts = pref[NPAIR - 1:NPAIR, :]
    pc = jnp.floor((counts + (BLK - 1)) * (1.0 / BLK)) * BLK
    cpc = pc
    s = 1
    while s < E:
        cpc = cpc + jnp.concatenate(
            [jnp.zeros((1, s), jnp.float32), cpc[:, :-s]], axis=1)
        s *= 2
    ends = cpc
    offs = cpc - pc

    rank = jnp.sum(onehot * pref, axis=1, keepdims=True)
    obase = jnp.sum(onehot * offs, axis=1, keepdims=True)
    pos = (obase + rank - 1.0).astype(jnp.int32)

    jio = lax.broadcasted_iota(jnp.int32, (128, 1), 0) * BLK
    bexp = jnp.sum((ends.astype(jnp.int32) <= jio).astype(jnp.int32),
                   axis=1, keepdims=True)
    bexp = jnp.minimum(bexp, E - 1)
    total = ends[0:1, E - 1:E]
    nact = (total * (1.0 / BLK)).astype(jnp.int32)
    meta = jnp.concatenate(
        [bexp, jnp.broadcast_to(nact, (128, 1)).astype(jnp.int32)], axis=1)

    pos_ref[...] = pos
    wp_ref[...] = w_pair
    meta_ref[...] = meta


def _router(xf, router_w):
    return pl.pallas_call(
        _router_body,
        out_shape=[
            jax.ShapeDtypeStruct((NPAIR, 1), jnp.int32),
            jax.ShapeDtypeStruct((NPAIR, 1), jnp.float32),
            jax.ShapeDtypeStruct((128, 2), jnp.int32),
        ],
    )(xf, router_w)


def _dispatch_body(x_hbm, pos_hbm, w_hbm, xs_out, wt_out,
                   idx_v, w_v, tok_v, rows_v, sem, sem2, sem3):
    wid = lax.axis_index("s") * 2 + lax.axis_index("c")
    base = wid * PAIRS_W
    m0 = pltpu.async_copy(pos_hbm.at[pl.ds(wid * 2, 2)], idx_v, sem3)
    m1 = pltpu.async_copy(w_hbm.at[pl.ds(wid * 2, 2)], w_v, sem3)
    for c in range(PAIRS_W // 16):
        lane = lax.iota(jnp.int32, 16)
        tok_v[pl.ds(c * 16, 16)] = (lane + (base + c * 16)) & (T - 1)
    g0 = pltpu.async_copy(
        x_hbm.at[tok_v.at[pl.ds(0, 64)]], rows_v.at[pl.ds(0, 64)], sem)
    g1 = pltpu.async_copy(
        x_hbm.at[tok_v.at[pl.ds(64, 64)]], rows_v.at[pl.ds(64, 64)], sem2)
    m0.wait()
    m1.wait()
    g0.wait()
    s0 = pltpu.async_copy(rows_v.at[pl.ds(0, 64)],
                          xs_out.at[idx_v.at[0]], sem)
    s0w = pltpu.async_copy(w_v.at[0], wt_out.at[idx_v.at[0]], sem)
    g1.wait()
    s1 = pltpu.async_copy(rows_v.at[pl.ds(64, 64)],
                          xs_out.at[idx_v.at[1]], sem2)
    s1w = pltpu.async_copy(w_v.at[1], wt_out.at[idx_v.at[1]], sem2)
    s0.wait()
    s0w.wait()
    s1.wait()
    s1w.wait()


def _dispatch(xf, pos2d, w2d):
    mesh = plsc.VectorSubcoreMesh(core_axis_name="c", subcore_axis_name="s",
                                  num_cores=2, num_subcores=16)
    return pl.kernel(
        _dispatch_body,
        out_type=[
            jax.ShapeDtypeStruct((NPAD, H2), jnp.int32),
            jax.ShapeDtypeStruct((NPAD,), jnp.float32),
        ],
        mesh=mesh,
        scratch_types=[
            pltpu.VMEM((2, 64), jnp.int32),
            pltpu.VMEM((2, 64), jnp.float32),
            pltpu.VMEM((PAIRS_W,), jnp.int32),
            pltpu.VMEM((PAIRS_W, H2), jnp.int32),
            pltpu.SemaphoreType.DMA,
            pltpu.SemaphoreType.DMA,
            pltpu.SemaphoreType.DMA,
        ],
    )(xf, pos2d, w2d)


def _ffn_body(bexp_ref, nact_ref, xs_ref, w1a_ref, w1b_ref, w2a_ref, w2b_ref,
              wt_ref, out_ref):
    j = pl.program_id(0)

    @pl.when(j < nact_ref[0])
    def _go():
        xw = xs_ref[...]
        xlo = lax.bitcast_convert_type(xw << 16, jnp.float32)
        xhi = lax.bitcast_convert_type(
            xw & jnp.int32(-65536), jnp.float32)
        xg = jnp.concatenate([xlo, xhi], axis=1)
        dn = (((1,), (1,)), ((), ()))
        h = jnp.concatenate(
            [lax.dot_general(xg, w1a_ref[0], dn,
                             preferred_element_type=jnp.float32),
             lax.dot_general(xg, w1b_ref[0], dn,
                             preferred_element_type=jnp.float32)],
            axis=1)
        g = h * 0.5 * (1.0 + lax.erf(h * 0.7071067811865476))
        y = jnp.concatenate(
            [lax.dot_general(g, w2a_ref[0], dn,
                             preferred_element_type=jnp.float32),
             lax.dot_general(g, w2b_ref[0], dn,
                             preferred_element_type=jnp.float32)],
            axis=1)
        out_ref[...] = y * wt_ref[...]


def _ffn(bexp, nact, xs, w1, w2, wt_col):
    grid_spec = pltpu.PrefetchScalarGridSpec(
        num_scalar_prefetch=2,
        grid=(NBLK,),
        in_specs=[
            pl.BlockSpec((BLK, H2),
                         lambda j, bexp, nact: (jnp.minimum(j, nact[0] - 1), 0)),
            pl.BlockSpec((1, F // 2, H), lambda j, bexp, nact: (bexp[j], 0, 0)),
            pl.BlockSpec((1, F // 2, H), lambda j, bexp, nact: (bexp[j], 1, 0)),
            pl.BlockSpec((1, H // 2, F), lambda j, bexp, nact: (bexp[j], 0, 0)),
            pl.BlockSpec((1, H // 2, F), lambda j, bexp, nact: (bexp[j], 1, 0)),
            pl.BlockSpec((BLK, 1),
                         lambda j, bexp, nact: (jnp.minimum(j, nact[0] - 1), 0)),
        ],
        out_specs=pl.BlockSpec(
            (BLK, H), lambda j, bexp, nact: (jnp.minimum(j, nact[0] - 1), 0)),
    )
    return pl.pallas_call(
        _ffn_body,
        grid_spec=grid_spec,
        out_shape=jax.ShapeDtypeStruct((NPAD, H), jnp.float32),
    )(bexp, nact, xs, w1, w1, w2, w2, wt_col)


def _combine_body(ys_hbm, pos_hbm, out_hbm, idx_v, a_v, b_v, sem):
    wid = lax.axis_index("s") * 2 + lax.axis_index("c")
    tbase = wid * TOK_W
    pltpu.sync_copy(pos_hbm.at[pl.ds(wid, 1)], idx_v.at[pl.ds(0, 1)])
    pltpu.sync_copy(pos_hbm.at[pl.ds(NW + wid, 1)], idx_v.at[pl.ds(1, 1)])
    g0 = pltpu.async_copy(ys_hbm.at[idx_v.at[0]], a_v, sem)
    g1 = pltpu.async_copy(ys_hbm.at[idx_v.at[1]], b_v, sem)
    g0.wait()
    g1.wait()

    def row(r, carry):
        for c in range(H // 16):
            sl = (r, pl.ds(c * 16, 16))
            a_v[sl] = a_v[sl] + b_v[sl]
        return carry

    lax.fori_loop(0, TOK_W, row, 0)
    pltpu.sync_copy(a_v, out_hbm.at[pl.ds(tbase, TOK_W)])


def _combine(ys, pos2d):
    mesh = plsc.VectorSubcoreMesh(core_axis_name="c", subcore_axis_name="s",
                                  num_cores=2, num_subcores=16)
    return pl.kernel(
        _combine_body,
        out_type=jax.ShapeDtypeStruct((T, H), jnp.float32),
        mesh=mesh,
        scratch_types=[
            pltpu.VMEM((2, 64), jnp.int32),
            pltpu.VMEM((TOK_W, H), jnp.float32),
            pltpu.VMEM((TOK_W, H), jnp.float32),
            pltpu.SemaphoreType.DMA,
        ],
    )(ys, pos2d)


def kernel(x, router_w, expert_w1, expert_w2):
    Bz, Sz, Hz = x.shape
    xf = x.reshape(T, H)
    pos, wp, meta = _router(xf, router_w)
    pos2d = pos.reshape(NPAIR // 64, 64)
    w2d = wp.reshape(NPAIR // 64, 64)
    bexp = meta[:, 0]
    nact = meta[0:1, 1]
    xb = xf.astype(jnp.bfloat16)
    lo16 = lax.bitcast_convert_type(xb[:, :H2], jnp.uint16).astype(jnp.uint32)
    hi16 = lax.bitcast_convert_type(xb[:, H2:], jnp.uint16).astype(jnp.uint32)
    xb32 = (lo16 | (hi16 << 16)).astype(jnp.int32)
    xs, wt = _dispatch(xb32, pos2d, w2d)
    ys = _ffn(bexp, nact, xs, expert_w1, expert_w2, wt.reshape(NPAD, 1))
    out = _combine(ys, pos2d)
    return out.reshape(Bz, Sz, Hz)


# speedup vs baseline: 13.6922x; 1.0224x over previous
import functools

import jax
import jax.numpy as jnp
from jax import lax
from jax.experimental import pallas as pl
from jax.experimental.pallas import tpu as pltpu
from jax.experimental.pallas import tpu_sc as plsc

T = 2048
H = 768
E = 64
K = 2
F = 2048
BLK = 128
NPAIR = T * K
NBLK = 96
NPAD = NBLK * BLK
H2 = H // 2
NW = 32
PAIRS_W = NPAIR // NW
TOK_W = T // NW


def _router_body(x_ref, rw_ref, pos_ref, wp_ref, meta_ref, xpk_ref):
    x = x_ref[...]
    rw = rw_ref[...]
    scores = lax.dot_general(
        x, rw, (((1,), (1,)), ((), ())),
        preferred_element_type=jnp.float32,
        precision=lax.Precision.DEFAULT)

    eio = lax.broadcasted_iota(jnp.int32, (T, E), 1)
    s1 = jnp.max(scores, axis=1, keepdims=True)
    a1 = jnp.min(jnp.where(scores == s1, eio, E), axis=1, keepdims=True)
    masked = jnp.where(eio == a1, jnp.float32(-jnp.inf), scores)
    s2 = jnp.max(masked, axis=1, keepdims=True)
    a2 = jnp.min(jnp.where(masked == s2, eio, E), axis=1, keepdims=True)
    ee = jnp.exp(s2 - s1)
    wa = 1.0 / (1.0 + ee)
    wb = ee / (1.0 + ee)

    e_pair = jnp.concatenate([a1, a2], axis=0)
    w_pair = jnp.concatenate([wa, wb], axis=0)

    pio = lax.broadcasted_iota(jnp.int32, (NPAIR, E), 1)
    onehot = (e_pair == pio).astype(jnp.float32)
    pref = onehot
    s = 1
    while s < NPAIR:
        pref = pref + jnp.concatenate(
            [jnp.zeros((s, E), jnp.float32), pref[:-s, :]], axis=0)
        s *= 2
    counts = pref[NPAIR - 1:NPAIR, :]
    pc = jnp.floor((counts + (BLK - 1)) * (1.0 / BLK)) * BLK
    cpc = pc
    s = 1
    while s < E:
        cpc = cpc + jnp.concatenate(
            [jnp.zeros((1, s), jnp.float32), cpc[:, :-s]], axis=1)
        s *= 2
    ends = cpc
    offs = cpc - pc

    rank = jnp.sum(onehot * pref, axis=1, keepdims=True)
    obase = jnp.sum(onehot * offs, axis=1, keepdims=True)
    pos = (obase + rank - 1.0).astype(jnp.int32)

    jio = lax.broadcasted_iota(jnp.int32, (128, 1), 0) * BLK
    bexp = jnp.sum((ends.astype(jnp.int32) <= jio).astype(jnp.int32),
                   axis=1, keepdims=True)
    bexp = jnp.minimum(bexp, E - 1)
    total = ends[0:1, E - 1:E]
    nact = (total * (1.0 / BLK)).astype(jnp.int32)
    meta = jnp.concatenate(
        [bexp, jnp.broadcast_to(nact, (128, 1)).astype(jnp.int32)], axis=1)

    pos_ref[...] = pos
    wp_ref[...] = w_pair
    meta_ref[...] = meta

    blo = lax.bitcast_convert_type(x[:, :H2], jnp.int32)
    bhi = lax.bitcast_convert_type(x[:, H2:], jnp.int32)
    rlo = blo + 0x7FFF + ((blo >> 16) & 1)
    rhi = bhi + 0x7FFF + ((bhi >> 16) & 1)
    xpk_ref[...] = (lax.shift_right_logical(rlo, jnp.int32(16))
                    | (rhi & jnp.int32(-65536)))


def _router(xf, router_w):
    return pl.pallas_call(
        _router_body,
        out_shape=[
            jax.ShapeDtypeStruct((NPAIR, 1), jnp.int32),
            jax.ShapeDtypeStruct((NPAIR, 1), jnp.float32),
            jax.ShapeDtypeStruct((128, 2), jnp.int32),
            jax.ShapeDtypeStruct((T, H2), jnp.int32),
        ],
    )(xf, router_w)


def _dispatch_body(x_hbm, pos_hbm, w_hbm, xs_out, wt_out,
                   idx_v, w_v, tok_v, rows_v, sem, sem2, sem3):
    wid = lax.axis_index("s") * 2 + lax.axis_index("c")
    base = wid * PAIRS_W
    m0 = pltpu.async_copy(pos_hbm.at[pl.ds(wid * 2, 2)], idx_v, sem3)
    m1 = pltpu.async_copy(w_hbm.at[pl.ds(wid * 2, 2)], w_v, sem3)
    for c in range(PAIRS_W // 16):
        lane = lax.iota(jnp.int32, 16)
        tok_v[pl.ds(c * 16, 16)] = (lane + (base + c * 16)) & (T - 1)
    g0 = pltpu.async_copy(
        x_hbm.at[tok_v.at[pl.ds(0, 64)]], rows_v.at[pl.ds(0, 64)], sem)
    g1 = pltpu.async_copy(
        x_hbm.at[tok_v.at[pl.ds(64, 64)]], rows_v.at[pl.ds(64, 64)], sem2)
    m0.wait()
    m1.wait()
    g0.wait()
    s0 = pltpu.async_copy(rows_v.at[pl.ds(0, 64)],
                          xs_out.at[idx_v.at[0]], sem)
    s0w = pltpu.async_copy(w_v.at[0], wt_out.at[idx_v.at[0]], sem)
    g1.wait()
    s1 = pltpu.async_copy(rows_v.at[pl.ds(64, 64)],
                          xs_out.at[idx_v.at[1]], sem2)
    s1w = pltpu.async_copy(w_v.at[1], wt_out.at[idx_v.at[1]], sem2)
    s0.wait()
    s0w.wait()
    s1.wait()
    s1w.wait()


def _dispatch(xf, pos2d, w2d):
    mesh = plsc.VectorSubcoreMesh(core_axis_name="c", subcore_axis_name="s",
                                  num_cores=2, num_subcores=16)
    return pl.kernel(
        _dispatch_body,
        out_type=[
            jax.ShapeDtypeStruct((NPAD, H2), jnp.int32),
            jax.ShapeDtypeStruct((NPAD,), jnp.float32),
        ],
        mesh=mesh,
        scratch_types=[
            pltpu.VMEM((2, 64), jnp.int32),
            pltpu.VMEM((2, 64), jnp.float32),
            pltpu.VMEM((PAIRS_W,), jnp.int32),
            pltpu.VMEM((PAIRS_W, H2), jnp.int32),
            pltpu.SemaphoreType.DMA,
            pltpu.SemaphoreType.DMA,
            pltpu.SemaphoreType.DMA,
        ],
    )(xf, pos2d, w2d)


def _ffn_body(bexp_ref, nact_ref, xs_ref, w1_ref, w2_ref, wt_ref, out_ref):
    j = pl.program_id(0)

    @pl.when(j < nact_ref[0])
    def _go():
        xw = xs_ref[...]
        xlo = lax.bitcast_convert_type(xw << 16, jnp.float32)
        xhi = lax.bitcast_convert_type(
            xw & jnp.int32(-65536), jnp.float32)
        xg = jnp.concatenate([xlo, xhi], axis=1)
        h = lax.dot_general(
            xg, w1_ref[0], (((1,), (1,)), ((), ())),
            preferred_element_type=jnp.float32)
        g = h * 0.5 * (1.0 + lax.erf(h * 0.7071067811865476))
        y = lax.dot_general(
            g, w2_ref[0], (((1,), (1,)), ((), ())),
            preferred_element_type=jnp.float32)
        out_ref[...] = y * wt_ref[...]


def _ffn(bexp, nact, xs, w1, w2, wt_col):
    grid_spec = pltpu.PrefetchScalarGridSpec(
        num_scalar_prefetch=2,
        grid=(NBLK,),
        in_specs=[
            pl.BlockSpec((BLK, H2),
                         lambda j, bexp, nact: (jnp.minimum(j, nact[0] - 1), 0)),
            pl.BlockSpec((1, F, H), lambda j, bexp, nact: (bexp[j], 0, 0)),
            pl.BlockSpec((1, H, F), lambda j, bexp, nact: (bexp[j], 0, 0)),
            pl.BlockSpec((BLK, 1),
                         lambda j, bexp, nact: (jnp.minimum(j, nact[0] - 1), 0)),
        ],
        out_specs=pl.BlockSpec(
            (BLK, H), lambda j, bexp, nact: (jnp.minimum(j, nact[0] - 1), 0)),
    )
    return pl.pallas_call(
        _ffn_body,
        grid_spec=grid_spec,
        out_shape=jax.ShapeDtypeStruct((NPAD, H), jnp.float32),
    )(bexp, nact, xs, w1, w2, wt_col)


def _combine_body(ys_hbm, pos_hbm, out_hbm, idx_v, a_v, b_v, sem):
    wid = lax.axis_index("s") * 2 + lax.axis_index("c")
    tbase = wid * TOK_W
    pltpu.sync_copy(pos_hbm.at[pl.ds(wid, 1)], idx_v.at[pl.ds(0, 1)])
    pltpu.sync_copy(pos_hbm.at[pl.ds(NW + wid, 1)], idx_v.at[pl.ds(1, 1)])
    g0 = pltpu.async_copy(ys_hbm.at[idx_v.at[0]], a_v, sem)
    g1 = pltpu.async_copy(ys_hbm.at[idx_v.at[1]], b_v, sem)
    g0.wait()
    g1.wait()

    def row(r, carry):
        for c in range(H // 16):
            sl = (r, pl.ds(c * 16, 16))
            a_v[sl] = a_v[sl] + b_v[sl]
        return carry

    lax.fori_loop(0, TOK_W, row, 0)
    pltpu.sync_copy(a_v, out_hbm.at[pl.ds(tbase, TOK_W)])


def _combine(ys, pos2d):
    mesh = plsc.VectorSubcoreMesh(core_axis_name="c", subcore_axis_name="s",
                                  num_cores=2, num_subcores=16)
    return pl.kernel(
        _combine_body,
        out_type=jax.ShapeDtypeStruct((T, H), jnp.float32),
        mesh=mesh,
        scratch_types=[
            pltpu.VMEM((2, 64), jnp.int32),
            pltpu.VMEM((TOK_W, H), jnp.float32),
            pltpu.VMEM((TOK_W, H), jnp.float32),
            pltpu.SemaphoreType.DMA,
        ],
    )(ys, pos2d)


def kernel(x, router_w, expert_w1, expert_w2):
    Bz, Sz, Hz = x.shape
    xf = x.reshape(T, H)
    pos, wp, meta, xb32 = _router(xf, router_w)
    pos2d = pos.reshape(NPAIR // 64, 64)
    w2d = wp.reshape(NPAIR // 64, 64)
    bexp = meta[:, 0]
    nact = meta[0:1, 1]
    xs, wt = _dispatch(xb32, pos2d, w2d)
    ys = _ffn(bexp, nact, xs, expert_w1, expert_w2, wt.reshape(NPAD, 1))
    out = _combine(ys, pos2d)
    return out.reshape(Bz, Sz, Hz)


# speedup vs baseline: 13.7514x; 1.0043x over previous
import jax
import jax.numpy as jnp
from jax import lax
from jax.experimental import pallas as pl
from jax.experimental.pallas import tpu as pltpu
from jax.experimental.pallas import tpu_sc as plsc

T = 2048
H = 768
E = 64
K = 2
F = 2048
BLK = 128
NPAIR = T * K
NBLK = 96
NPAD = NBLK * BLK
H2 = H // 2
NW = 32
PAIRS_W = NPAIR // NW
TOK_W = T // NW


def _router_body(x_ref, rw_ref, pos_ref, wp_ref, meta_ref, xpk_ref):
    x = x_ref[...]
    rw = rw_ref[...]
    scores = lax.dot_general(
        x, rw, (((1,), (1,)), ((), ())),
        preferred_element_type=jnp.float32,
        precision=lax.Precision.DEFAULT)

    eio = lax.broadcasted_iota(jnp.int32, (T, E), 1)
    s1 = jnp.max(scores, axis=1, keepdims=True)
    a1 = jnp.min(jnp.where(scores == s1, eio, E), axis=1, keepdims=True)
    masked = jnp.where(eio == a1, jnp.float32(-jnp.inf), scores)
    s2 = jnp.max(masked, axis=1, keepdims=True)
    a2 = jnp.min(jnp.where(masked == s2, eio, E), axis=1, keepdims=True)
    ee = jnp.exp(s2 - s1)
    wa = 1.0 / (1.0 + ee)
    wb = ee / (1.0 + ee)

    e_pair = jnp.concatenate([a1, a2], axis=0)
    w_pair = jnp.concatenate([wa, wb], axis=0)

    pio = lax.broadcasted_iota(jnp.int32, (NPAIR, E), 1)
    onehot = (e_pair == pio).astype(jnp.float32)
    pref = onehot
    s = 1
    while s < NPAIR:
        pref = pref + jnp.concatenate(
            [jnp.zeros((s, E), jnp.float32), pref[:-s, :]], axis=0)
        s *= 2
    counts = pref[NPAIR - 1:NPAIR, :]
    pc = jnp.floor((counts + (BLK - 1)) * (1.0 / BLK)) * BLK
    cpc = pc
    s = 1
    while s < E:
        cpc = cpc + jnp.concatenate(
            [jnp.zeros((1, s), jnp.float32), cpc[:, :-s]], axis=1)
        s *= 2
    ends = cpc
    offs = cpc - pc

    rank = jnp.sum(onehot * pref, axis=1, keepdims=True)
    obase = jnp.sum(onehot * offs, axis=1, keepdims=True)
    pos = (obase + rank - 1.0).astype(jnp.int32)

    jio = lax.broadcasted_iota(jnp.int32, (128, 1), 0) * BLK
    bexp = jnp.sum((ends.astype(jnp.int32) <= jio).astype(jnp.int32),
                   axis=1, keepdims=True)
    bexp = jnp.minimum(bexp, E - 1)
    total = ends[0:1, E - 1:E]
    nact = (total * (1.0 / BLK)).astype(jnp.int32)
    meta = jnp.concatenate(
        [bexp, jnp.broadcast_to(nact, (128, 1)).astype(jnp.int32)], axis=1)

    pos_ref[...] = pos
    wp_ref[...] = w_pair
    meta_ref[...] = meta

    blo = lax.bitcast_convert_type(x[:, :H2], jnp.int32)
    bhi = lax.bitcast_convert_type(x[:, H2:], jnp.int32)
    rlo = blo + 0x7FFF + ((blo >> 16) & 1)
    rhi = bhi + 0x7FFF + ((bhi >> 16) & 1)
    xpk_ref[...] = (lax.shift_right_logical(rlo, jnp.int32(16))
                    | (rhi & jnp.int32(-65536)))


def _router(xf, router_w):
    return pl.pallas_call(
        _router_body,
        out_shape=[
            jax.ShapeDtypeStruct((NPAIR, 1), jnp.int32),
            jax.ShapeDtypeStruct((NPAIR, 1), jnp.float32),
            jax.ShapeDtypeStruct((128, 2), jnp.int32),
            jax.ShapeDtypeStruct((T, H2), jnp.int32),
        ],
    )(xf, router_w)


def _dispatch_body(x_hbm, pos_hbm, w_hbm, xs_out, wt_out,
                   idx_v, w_v, tok_v, rows_v, sem, sem2, sem3):
    wid = lax.axis_index("s") * 2 + lax.axis_index("c")
    base = wid * PAIRS_W
    m0 = pltpu.async_copy(pos_hbm.at[pl.ds(wid * 2, 2)], idx_v, sem3)
    m1 = pltpu.async_copy(w_hbm.at[pl.ds(wid * 2, 2)], w_v, sem3)
    for c in range(PAIRS_W // 16):
        lane = lax.iota(jnp.int32, 16)
        tok_v[pl.ds(c * 16, 16)] = (lane + (base + c * 16)) & (T - 1)
    g0 = pltpu.async_copy(
        x_hbm.at[tok_v.at[pl.ds(0, 64)]], rows_v.at[pl.ds(0, 64)], sem)
    g1 = pltpu.async_copy(
        x_hbm.at[tok_v.at[pl.ds(64, 64)]], rows_v.at[pl.ds(64, 64)], sem2)
    m0.wait()
    m1.wait()
    g0.wait()
    s0 = pltpu.async_copy(rows_v.at[pl.ds(0, 64)],
                          xs_out.at[idx_v.at[0]], sem)
    s0w = pltpu.async_copy(w_v.at[0], wt_out.at[idx_v.at[0]], sem)
    g1.wait()
    s1 = pltpu.async_copy(rows_v.at[pl.ds(64, 64)],
                          xs_out.at[idx_v.at[1]], sem2)
    s1w = pltpu.async_copy(w_v.at[1], wt_out.at[idx_v.at[1]], sem2)
    s0.wait()
    s0w.wait()
    s1.wait()
    s1w.wait()


def _dispatch(xf, pos2d, w2d):
    mesh = plsc.VectorSubcoreMesh(core_axis_name="c", subcore_axis_name="s",
                                  num_cores=2, num_subcores=16)
    return pl.kernel(
        _dispatch_body,
        out_type=[
            jax.ShapeDtypeStruct((NPAD, H2), jnp.int32),
            jax.ShapeDtypeStruct((NPAD,), jnp.float32),
        ],
        mesh=mesh,
        scratch_types=[
            pltpu.VMEM((2, 64), jnp.int32),
            pltpu.VMEM((2, 64), jnp.float32),
            pltpu.VMEM((PAIRS_W,), jnp.int32),
            pltpu.VMEM((PAIRS_W, H2), jnp.int32),
            pltpu.SemaphoreType.DMA,
            pltpu.SemaphoreType.DMA,
            pltpu.SemaphoreType.DMA,
        ],
    )(xf, pos2d, w2d)


def _ffn_body(bexp_ref, nact_ref, xs_ref, w1_ref, w2_ref, wt_ref, out_ref):
    j = pl.program_id(0)

    @pl.when(j < nact_ref[0])
    def _go():
        xw = xs_ref[...]
        xlo = lax.bitcast_convert_type(xw << 16, jnp.float32)
        xhi = lax.bitcast_convert_type(
            xw & jnp.int32(-65536), jnp.float32)
        xg = jnp.concatenate([xlo, xhi], axis=1)
        h = lax.dot_general(
            xg, w1_ref[0], (((1,), (1,)), ((), ())),
            preferred_element_type=jnp.float32)
        g = h * 0.5 * (1.0 + lax.erf(h * 0.7071067811865476))
        y = lax.dot_general(
            g, w2_ref[0], (((1,), (1,)), ((), ())),
            preferred_element_type=jnp.float32)
        out_ref[...] = y * wt_ref[...]


def _ffn(bexp, nact, xs, w1, w2, wt_col):
    grid_spec = pltpu.PrefetchScalarGridSpec(
        num_scalar_prefetch=2,
        grid=(NBLK,),
        in_specs=[
            pl.BlockSpec((BLK, H2),
                         lambda j, bexp, nact: (jnp.minimum(j, nact[0] - 1), 0)),
            pl.BlockSpec((1, F, H), lambda j, bexp, nact: (bexp[j], 0, 0)),
            pl.BlockSpec((1, H, F), lambda j, bexp, nact: (bexp[j], 0, 0)),
            pl.BlockSpec((BLK, 1),
                         lambda j, bexp, nact: (jnp.minimum(j, nact[0] - 1), 0)),
        ],
        out_specs=pl.BlockSpec(
            (BLK, H), lambda j, bexp, nact: (jnp.minimum(j, nact[0] - 1), 0)),
    )
    return pl.pallas_call(
        _ffn_body,
        grid_spec=grid_spec,
        out_shape=jax.ShapeDtypeStruct((NPAD, H), jnp.float32),
    )(bexp, nact, xs, w1, w2, wt_col)


def _combine_body(ys_hbm, pos_hbm, out_hbm, idx_v, a_v, b_v, sem):
    wid = lax.axis_index("s") * 2 + lax.axis_index("c")
    tbase = wid * TOK_W
    pltpu.sync_copy(pos_hbm.at[pl.ds(wid, 1)], idx_v.at[pl.ds(0, 1)])
    pltpu.sync_copy(pos_hbm.at[pl.ds(NW + wid, 1)], idx_v.at[pl.ds(1, 1)])
    g0 = pltpu.async_copy(ys_hbm.at[idx_v.at[0]], a_v, sem)
    g1 = pltpu.async_copy(ys_hbm.at[idx_v.at[1]], b_v, sem)
    g0.wait()
    g1.wait()

    def row(r, carry):
        for c in range(H // 16):
            sl = (r, pl.ds(c * 16, 16))
            a_v[sl] = a_v[sl] + b_v[sl]
        return carry

    lax.fori_loop(0, TOK_W, row, 0)
    pltpu.sync_copy(a_v, out_hbm.at[pl.ds(tbase, TOK_W)])


def _combine(ys, pos2d):
    mesh = plsc.VectorSubcoreMesh(core_axis_name="c", subcore_axis_name="s",
                                  num_cores=2, num_subcores=16)
    return pl.kernel(
        _combine_body,
        out_type=jax.ShapeDtypeStruct((T, H), jnp.float32),
        mesh=mesh,
        scratch_types=[
            pltpu.VMEM((2, 64), jnp.int32),
            pltpu.VMEM((TOK_W, H), jnp.float32),
            pltpu.VMEM((TOK_W, H), jnp.float32),
            pltpu.SemaphoreType.DMA,
        ],
    )(ys, pos2d)


def kernel(x, router_w, expert_w1, expert_w2):
    Bz, Sz, Hz = x.shape
    xf = x.reshape(T, H)
    pos, wp, meta, xb32 = _router(xf, router_w)
    pos2d = pos.reshape(NPAIR // 64, 64)
    w2d = wp.reshape(NPAIR // 64, 64)
    bexp = meta[:, 0]
    nact = meta[0:1, 1]
    xs, wt = _dispatch(xb32, pos2d, w2d)
    ys = _ffn(bexp, nact, xs, expert_w1, expert_w2, wt.reshape(NPAD, 1))
    out = _combine(ys, pos2d)
    return out.reshape(Bz, Sz, Hz)
